# baseline (speedup 1.0000x reference)
import functools

import jax
import jax.numpy as jnp
from jax import lax
from jax.experimental import pallas as pl
from jax.experimental.pallas import tpu as pltpu
from jax.experimental.pallas import tpu_sc as plsc

_S = 20
_B = 4096
_H = 128

_NC = 2
_NS = 16
_NW = _NC * _NS
_GCHUNK = 128


def _sc_gather(idx3, table):
    nw, cpw, g = idx3.shape
    h = table.shape[1]
    total = nw * cpw * g
    mesh = plsc.VectorSubcoreMesh(core_axis_name="c", subcore_axis_name="s")

    @functools.partial(
        pl.kernel,
        mesh=mesh,
        out_type=jax.ShapeDtypeStruct((total, h), jnp.float32),
        scratch_types=[
            pltpu.VMEM((cpw, g), jnp.int32),
            pltpu.VMEM((2, g, h), jnp.float32),
            pltpu.SemaphoreType.DMA,
            pltpu.SemaphoreType.DMA,
            pltpu.SemaphoreType.DMA,
        ],
    )
    def gather_kernel(idx_hbm, table_hbm, out_hbm, idx_v, rows_v, gsem0, gsem1, osem):
        wid = lax.axis_index("s") * _NC + lax.axis_index("c")
        base = wid * (cpw * g)
        pltpu.sync_copy(idx_hbm.at[wid], idx_v)
        gsems = (gsem0, gsem1)
        cp = pltpu.async_copy(table_hbm.at[idx_v.at[0]], rows_v.at[0], gsems[0])
        prev_out = None
        for j in range(cpw):
            cur = j % 2
            if j + 1 < cpw:
                nxt = pltpu.async_copy(
                    table_hbm.at[idx_v.at[j + 1]], rows_v.at[1 - cur], gsems[1 - cur]
                )
            cp.wait()
            if prev_out is not None:
                prev_out.wait()
            prev_out = pltpu.async_copy(
                rows_v.at[cur], out_hbm.at[pl.ds(base + j * g, g)], osem
            )
            if j + 1 < cpw:
                cp = nxt
        prev_out.wait()

    return gather_kernel(idx3, table)


def _gru_body(emb_ref, h0_ref, wih_ref, whh_ref, bih_ref, bhh_ref,
              out_ref, hfin_ref, h_scr):
    s = pl.program_id(1)

    @pl.when(s == 0)
    def _():
        h_scr[...] = h0_ref[0]

    xt = emb_ref[0].astype(jnp.bfloat16)
    h = h_scr[...]
    hb = h.astype(jnp.bfloat16)
    gi = jnp.dot(xt, wih_ref[...], preferred_element_type=jnp.float32) + bih_ref[...]
    gh = jnp.dot(hb, whh_ref[...], preferred_element_type=jnp.float32) + bhh_ref[...]
    r = 0.5 + 0.5 * jnp.tanh(0.5 * (gi[:, :_H] + gh[:, :_H]))
    z = 0.5 + 0.5 * jnp.tanh(0.5 * (gi[:, _H:2 * _H] + gh[:, _H:2 * _H]))
    n = jnp.tanh(gi[:, 2 * _H:] + r * gh[:, 2 * _H:])
    h_new = n + z * (h - n)
    h_scr[...] = h_new
    out_ref[0] = h_new

    @pl.when(s == _S - 1)
    def _():
        hfin_ref[0] = h_new


def _tc_gru(embedded, hidden, wih_t, whh_t, bih, bhh, block_b):
    nb = _B // block_b
    grid = (nb, _S)
    return pl.pallas_call(
        _gru_body,
        grid=grid,
        in_specs=[
            pl.BlockSpec((1, block_b, _H), lambda b, s: (s, b, 0)),
            pl.BlockSpec((1, block_b, _H), lambda b, s: (0, b, 0)),
            pl.BlockSpec((_H, 3 * _H), lambda b, s: (0, 0)),
            pl.BlockSpec((_H, 3 * _H), lambda b, s: (0, 0)),
            pl.BlockSpec((1, 3 * _H), lambda b, s: (0, 0)),
            pl.BlockSpec((1, 3 * _H), lambda b, s: (0, 0)),
        ],
        out_specs=[
            pl.BlockSpec((1, block_b, _H), lambda b, s: (s, b, 0)),
            pl.BlockSpec((1, block_b, _H), lambda b, s: (0, b, 0)),
        ],
        out_shape=[
            jax.ShapeDtypeStruct((_S, _B, _H), jnp.float32),
            jax.ShapeDtypeStruct((1, _B, _H), jnp.float32),
        ],
        scratch_shapes=[pltpu.VMEM((block_b, _H), jnp.float32)],
        compiler_params=pltpu.CompilerParams(
            dimension_semantics=("parallel", "arbitrary"),
        ),
    )(embedded, hidden, wih_t, whh_t, bih, bhh)


def kernel(x, hidden, emb, W_ih, W_hh, b_ih, b_hh):
    cpw = (_S * _B) // (_NW * _GCHUNK)
    idx3 = x.reshape(_NW, cpw, _GCHUNK)
    embedded_flat = _sc_gather(idx3, emb)
    embedded = embedded_flat.reshape(_S, _B, _H)
    outputs, h_final = _tc_gru(
        embedded, hidden,
        W_ih.T.astype(jnp.bfloat16), W_hh.T.astype(jnp.bfloat16),
        b_ih.reshape(1, -1), b_hh.reshape(1, -1),
        block_b=4096,
    )
    return outputs, h_final


# speedup vs baseline: 1.0825x; 1.0825x over previous
import functools

import jax
import jax.numpy as jnp
from jax import lax
from jax.experimental import pallas as pl
from jax.experimental.pallas import tpu as pltpu
from jax.experimental.pallas import tpu_sc as plsc

_S = 20
_B = 4096
_H = 128

_NC = 2
_NS = 16
_NW = _NC * _NS
_GCHUNK = 128


def _sc_gather(idx3, table):
    nw, cpw, g = idx3.shape
    h = table.shape[1]
    total = nw * cpw * g
    mesh = plsc.VectorSubcoreMesh(core_axis_name="c", subcore_axis_name="s")

    @functools.partial(
        pl.kernel,
        mesh=mesh,
        out_type=jax.ShapeDtypeStruct((total, h), jnp.float32),
        scratch_types=[
            pltpu.VMEM((cpw, g), jnp.int32),
            pltpu.VMEM((2, g, h), jnp.float32),
            pltpu.SemaphoreType.DMA,
            pltpu.SemaphoreType.DMA,
            pltpu.SemaphoreType.DMA,
        ],
    )
    def gather_kernel(idx_hbm, table_hbm, out_hbm, idx_v, rows_v, gsem0, gsem1, osem):
        wid = lax.axis_index("s") * _NC + lax.axis_index("c")
        base = wid * (cpw * g)
        pltpu.sync_copy(idx_hbm.at[wid], idx_v)
        gsems = (gsem0, gsem1)
        cp = pltpu.async_copy(table_hbm.at[idx_v.at[0]], rows_v.at[0], gsems[0])
        prev_out = None
        for j in range(cpw):
            cur = j % 2
            if j + 1 < cpw:
                nxt = pltpu.async_copy(
                    table_hbm.at[idx_v.at[j + 1]], rows_v.at[1 - cur], gsems[1 - cur]
                )
            cp.wait()
            if prev_out is not None:
                prev_out.wait()
            prev_out = pltpu.async_copy(
                rows_v.at[cur], out_hbm.at[pl.ds(base + j * g, g)], osem
            )
            if j + 1 < cpw:
                cp = nxt
        prev_out.wait()

    return gather_kernel(idx3, table)


def _gru_body(emb_ref, h0_ref, wrz_ref, win_ref, whn_ref, brz_ref,
              bin_ref, bhn_ref, out_ref, hfin_ref, h_scr):
    s = pl.program_id(1)

    @pl.when(s == 0)
    def _():
        h_scr[...] = h0_ref[0]

    xt = emb_ref[0].astype(jnp.bfloat16)
    h = h_scr[...]
    hb = h.astype(jnp.bfloat16)
    cat = jnp.concatenate([xt, hb], axis=1)
    rz = jnp.dot(cat, wrz_ref[...], preferred_element_type=jnp.float32) + brz_ref[...]
    i_n = jnp.dot(xt, win_ref[...], preferred_element_type=jnp.float32) + bin_ref[...]
    h_n = jnp.dot(hb, whn_ref[...], preferred_element_type=jnp.float32) + bhn_ref[...]
    r = 0.5 + 0.5 * jnp.tanh(0.5 * rz[:, :_H])
    z = 0.5 + 0.5 * jnp.tanh(0.5 * rz[:, _H:])
    n = jnp.tanh(i_n + r * h_n)
    h_new = n + z * (h - n)
    h_scr[...] = h_new
    out_ref[0] = h_new

    @pl.when(s == _S - 1)
    def _():
        hfin_ref[0] = h_new


def _tc_gru(embedded, hidden, wrz, win, whn, brz, bin_, bhn, block_b):
    nb = _B // block_b
    grid = (nb, _S)
    return pl.pallas_call(
        _gru_body,
        grid=grid,
        in_specs=[
            pl.BlockSpec((1, block_b, _H), lambda b, s: (s, b, 0)),
            pl.BlockSpec((1, block_b, _H), lambda b, s: (0, b, 0)),
            pl.BlockSpec((2 * _H, 2 * _H), lambda b, s: (0, 0)),
            pl.BlockSpec((_H, _H), lambda b, s: (0, 0)),
            pl.BlockSpec((_H, _H), lambda b, s: (0, 0)),
            pl.BlockSpec((1, 2 * _H), lambda b, s: (0, 0)),
            pl.BlockSpec((1, _H), lambda b, s: (0, 0)),
            pl.BlockSpec((1, _H), lambda b, s: (0, 0)),
        ],
        out_specs=[
            pl.BlockSpec((1, block_b, _H), lambda b, s: (s, b, 0)),
            pl.BlockSpec((1, block_b, _H), lambda b, s: (0, b, 0)),
        ],
        out_shape=[
            jax.ShapeDtypeStruct((_S, _B, _H), jnp.float32),
            jax.ShapeDtypeStruct((1, _B, _H), jnp.float32),
        ],
        scratch_shapes=[pltpu.VMEM((block_b, _H), jnp.float32)],
        compiler_params=pltpu.CompilerParams(
            dimension_semantics=("parallel", "arbitrary"),
        ),
    )(embedded, hidden, wrz, win, whn, brz, bin_, bhn)


def kernel(x, hidden, emb, W_ih, W_hh, b_ih, b_hh):
    cpw = (_S * _B) // (_NW * _GCHUNK)
    idx3 = x.reshape(_NW, cpw, _GCHUNK)
    embedded_flat = _sc_gather(idx3, emb)
    embedded = embedded_flat.reshape(_S, _B, _H)
    wih_t = W_ih.T
    whh_t = W_hh.T
    wrz = jnp.concatenate(
        [wih_t[:, :2 * _H], whh_t[:, :2 * _H]], axis=0).astype(jnp.bfloat16)
    win = wih_t[:, 2 * _H:].astype(jnp.bfloat16)
    whn = whh_t[:, 2 * _H:].astype(jnp.bfloat16)
    brz = (b_ih[:2 * _H] + b_hh[:2 * _H]).reshape(1, -1)
    bin_ = b_ih[2 * _H:].reshape(1, -1)
    bhn = b_hh[2 * _H:].reshape(1, -1)
    outputs, h_final = _tc_gru(
        embedded, hidden, wrz, win, whn, brz, bin_, bhn, block_b=4096)
    return outputs, h_final


# speedup vs baseline: 1.1141x; 1.0292x over previous
import functools

import jax
import jax.numpy as jnp
from jax import lax
from jax.experimental import pallas as pl
from jax.experimental.pallas import tpu as pltpu
from jax.experimental.pallas import tpu_sc as plsc

_S = 20
_B = 4096
_H = 128

_NC = 2
_NS = 16
_NW = _NC * _NS
_GCHUNK = 128


def _sc_gather(idx3, table):
    nw, cpw, g = idx3.shape
    h = table.shape[1]
    total = nw * cpw * g
    mesh = plsc.VectorSubcoreMesh(core_axis_name="c", subcore_axis_name="s")

    @functools.partial(
        pl.kernel,
        mesh=mesh,
        out_type=jax.ShapeDtypeStruct((total, h), jnp.float32),
        scratch_types=[
            pltpu.VMEM((cpw, g), jnp.int32),
            pltpu.VMEM((2, g, h), jnp.float32),
            pltpu.SemaphoreType.DMA,
            pltpu.SemaphoreType.DMA,
            pltpu.SemaphoreType.DMA,
        ],
    )
    def gather_kernel(idx_hbm, table_hbm, out_hbm, idx_v, rows_v, gsem0, gsem1, osem):
        wid = lax.axis_index("s") * _NC + lax.axis_index("c")
        base = wid * (cpw * g)
        pltpu.sync_copy(idx_hbm.at[wid], idx_v)
        gsems = (gsem0, gsem1)
        cp = pltpu.async_copy(table_hbm.at[idx_v.at[0]], rows_v.at[0], gsems[0])
        prev_out = None
        for j in range(cpw):
            cur = j % 2
            if j + 1 < cpw:
                nxt = pltpu.async_copy(
                    table_hbm.at[idx_v.at[j + 1]], rows_v.at[1 - cur], gsems[1 - cur]
                )
            cp.wait()
            if prev_out is not None:
                prev_out.wait()
            prev_out = pltpu.async_copy(
                rows_v.at[cur], out_hbm.at[pl.ds(base + j * g, g)], osem
            )
            if j + 1 < cpw:
                cp = nxt
        prev_out.wait()

    return gather_kernel(idx3, table)


_NCHUNK = 2
_SC_STEPS = _S // _NCHUNK


def _gru_body(nsteps, has_buf, *refs):
    if has_buf:
        (emb_ref, h0_ref, wrz_ref, win_ref, whn_ref, brz_ref,
         bin_ref, bhn_ref, _buf_ref, out_ref, hfin_ref, h_scr) = refs
    else:
        (emb_ref, h0_ref, wrz_ref, win_ref, whn_ref, brz_ref,
         bin_ref, bhn_ref, out_ref, hfin_ref, h_scr) = refs
    s = pl.program_id(1)

    @pl.when(s == 0)
    def _():
        h_scr[...] = h0_ref[0]

    xt = emb_ref[0].astype(jnp.bfloat16)
    h = h_scr[...]
    hb = h.astype(jnp.bfloat16)
    cat = jnp.concatenate([xt, hb], axis=1)
    rz = jnp.dot(cat, wrz_ref[...], preferred_element_type=jnp.float32) + brz_ref[...]
    i_n = jnp.dot(xt, win_ref[...], preferred_element_type=jnp.float32) + bin_ref[...]
    h_n = jnp.dot(hb, whn_ref[...], preferred_element_type=jnp.float32) + bhn_ref[...]
    r = 0.5 + 0.5 * jnp.tanh(0.5 * rz[:, :_H])
    z = 0.5 + 0.5 * jnp.tanh(0.5 * rz[:, _H:])
    n = jnp.tanh(i_n + r * h_n)
    h_new = n + z * (h - n)
    h_scr[...] = h_new
    out_ref[0] = h_new

    @pl.when(s == nsteps - 1)
    def _():
        hfin_ref[0] = h_new


def _tc_gru_chunk(embedded_c, buf, h_in, wrz, win, whn, brz, bin_, bhn,
                  step_off, block_b):
    nb = _B // block_b
    grid = (nb, _SC_STEPS)
    in_specs = [
        pl.BlockSpec((1, block_b, _H), lambda b, s: (s, b, 0)),
        pl.BlockSpec((1, block_b, _H), lambda b, s: (0, b, 0)),
        pl.BlockSpec((2 * _H, 2 * _H), lambda b, s: (0, 0)),
        pl.BlockSpec((_H, _H), lambda b, s: (0, 0)),
        pl.BlockSpec((_H, _H), lambda b, s: (0, 0)),
        pl.BlockSpec((1, 2 * _H), lambda b, s: (0, 0)),
        pl.BlockSpec((1, _H), lambda b, s: (0, 0)),
        pl.BlockSpec((1, _H), lambda b, s: (0, 0)),
    ]
    args = [embedded_c, h_in, wrz, win, whn, brz, bin_, bhn]
    aliases = {}
    if buf is not None:
        in_specs.append(pl.BlockSpec(memory_space=pl.ANY))
        args.append(buf)
        aliases = {8: 0}
    return pl.pallas_call(
        functools.partial(_gru_body, _SC_STEPS, buf is not None),
        grid=grid,
        in_specs=in_specs,
        out_specs=[
            pl.BlockSpec((1, block_b, _H), lambda b, s: (s + step_off, b, 0)),
            pl.BlockSpec((1, block_b, _H), lambda b, s: (0, b, 0)),
        ],
        out_shape=[
            jax.ShapeDtypeStruct((_S, _B, _H), jnp.float32),
            jax.ShapeDtypeStruct((1, _B, _H), jnp.float32),
        ],
        scratch_shapes=[pltpu.VMEM((block_b, _H), jnp.float32)],
        input_output_aliases=aliases,
        compiler_params=pltpu.CompilerParams(
            dimension_semantics=("parallel", "arbitrary"),
        ),
    )(*args)


def kernel(x, hidden, emb, W_ih, W_hh, b_ih, b_hh):
    cpw = (_SC_STEPS * _B) // (_NW * _GCHUNK)
    wih_t = W_ih.T
    whh_t = W_hh.T
    wrz = jnp.concatenate(
        [wih_t[:, :2 * _H], whh_t[:, :2 * _H]], axis=0).astype(jnp.bfloat16)
    win = wih_t[:, 2 * _H:].astype(jnp.bfloat16)
    whn = whh_t[:, 2 * _H:].astype(jnp.bfloat16)
    brz = (b_ih[:2 * _H] + b_hh[:2 * _H]).reshape(1, -1)
    bin_ = b_ih[2 * _H:].reshape(1, -1)
    bhn = b_hh[2 * _H:].reshape(1, -1)

    embedded_chunks = []
    for c in range(_NCHUNK):
        idx3 = x[c * _SC_STEPS:(c + 1) * _SC_STEPS].reshape(_NW, cpw, _GCHUNK)
        flat = _sc_gather(idx3, emb)
        embedded_chunks.append(flat.reshape(_SC_STEPS, _B, _H))

    buf = None
    h_cur = hidden
    for c in range(_NCHUNK):
        buf, h_cur = _tc_gru_chunk(
            embedded_chunks[c], buf, h_cur, wrz, win, whn, brz, bin_, bhn,
            step_off=c * _SC_STEPS, block_b=4096,
        )
    kept = lax.optimization_barrier((buf, h_cur, *embedded_chunks))
    return kept[0], kept[1]
